# baseline (speedup 1.0000x reference)
import functools

import jax
import jax.numpy as jnp
from jax import lax
from jax.experimental import pallas as pl
from jax.experimental.pallas import tpu as pltpu
from jax.experimental.pallas import tpu_sc as plsc

_NUM_SC = 2
_NUM_SUBCORES = 16
_NUM_WORKERS = _NUM_SC * _NUM_SUBCORES

VOCAB_TILE = 2048


def _sc_gather(table2, pair_idx):
    b, d2 = pair_idx.shape[0], table2.shape[1]
    b_per_w = b // _NUM_WORKERS
    mesh = plsc.VectorSubcoreMesh(core_axis_name="c", subcore_axis_name="s")

    @functools.partial(
        pl.kernel,
        mesh=mesh,
        out_type=jax.ShapeDtypeStruct((b, d2), jnp.float32),
        scratch_types=[
            pltpu.VMEM((b_per_w,), jnp.int32),
            pltpu.VMEM((b_per_w, d2), jnp.float32),
            pltpu.SemaphoreType.DMA,
        ],
    )
    def gather_kernel(table_hbm, idx_hbm, out_hbm, idx_v, rows_v, sem):
        wid = lax.axis_index("s") * _NUM_SC + lax.axis_index("c")
        base = wid * b_per_w
        pltpu.sync_copy(idx_hbm.at[pl.ds(base, b_per_w)], idx_v)
        pltpu.async_copy(table_hbm.at[idx_v], rows_v, sem).wait()
        pltpu.sync_copy(rows_v, out_hbm.at[pl.ds(base, b_per_w)])

    return gather_kernel(table2, pair_idx)


def _h_from_pairs(e2_ref, par_ref, emb):
    e2 = e2_ref[...]
    h = jnp.where(par_ref[...] != 0, e2[:, emb:], e2[:, :emb])
    return jnp.maximum(h, 0.0)


def _stats_body(vocab, emb, e2_ref, par_ref, w_ref, b_ref, logz_ref, m_s, s_s):
    j = pl.program_id(0)
    nt = pl.num_programs(0)

    @pl.when(j == 0)
    def _():
        m_s[...] = jnp.full(m_s.shape, -1e30, m_s.dtype)
        s_s[...] = jnp.zeros(s_s.shape, s_s.dtype)

    h = _h_from_pairs(e2_ref, par_ref, emb)
    logits = lax.dot_general(
        h, w_ref[...], (((1,), (1,)), ((), ())),
        preferred_element_type=jnp.float32) + b_ref[...]
    col = j * VOCAB_TILE + lax.broadcasted_iota(jnp.int32, (1, VOCAB_TILE), 1)
    logits = jnp.where(col < vocab, logits, -1e30)

    m_tile = jnp.max(logits, axis=1, keepdims=True)
    m_old = m_s[...]
    m_new = jnp.maximum(m_old, m_tile)
    s_s[...] = s_s[...] * jnp.exp(m_old - m_new) + jnp.sum(
        jnp.exp(logits - m_new), axis=1, keepdims=True)
    m_s[...] = m_new

    @pl.when(j == nt - 1)
    def _():
        logz_ref[...] = m_new + jnp.log(s_s[...])


def _write_body(emb, e2_ref, par_ref, w_ref, b_ref, logz_ref, out_ref):
    h = _h_from_pairs(e2_ref, par_ref, emb)
    logits = lax.dot_general(
        h, w_ref[...], (((1,), (1,)), ((), ())),
        preferred_element_type=jnp.float32) + b_ref[...]
    out_ref[...] = logits - logz_ref[...]


def kernel(input, table, W, b):
    bsz, emb = input.shape[0], table.shape[1]
    vocab = W.shape[0]

    idx = input.astype(jnp.int32)
    table2 = table.reshape(vocab // 2, 2 * emb)
    pairs = _sc_gather(table2, idx >> 1)
    par = (idx & 1).reshape(bsz, 1)

    b2 = b.reshape(1, vocab)
    nt = pl.cdiv(vocab, VOCAB_TILE)
    e2_spec = pl.BlockSpec((bsz, 2 * emb), lambda j: (0, 0))
    par_spec = pl.BlockSpec((bsz, 1), lambda j: (0, 0))
    w_spec = pl.BlockSpec((VOCAB_TILE, emb), lambda j: (j, 0))
    b_spec = pl.BlockSpec((1, VOCAB_TILE), lambda j: (0, j))
    z_spec = pl.BlockSpec((bsz, 1), lambda j: (0, 0))

    logz = pl.pallas_call(
        functools.partial(_stats_body, vocab, emb),
        grid=(nt,),
        in_specs=[e2_spec, par_spec, w_spec, b_spec],
        out_specs=z_spec,
        out_shape=jax.ShapeDtypeStruct((bsz, 1), jnp.float32),
        scratch_shapes=[
            pltpu.VMEM((bsz, 1), jnp.float32),
            pltpu.VMEM((bsz, 1), jnp.float32),
        ],
    )(pairs, par, W, b2)

    out = pl.pallas_call(
        functools.partial(_write_body, emb),
        grid=(nt,),
        in_specs=[e2_spec, par_spec, w_spec, b_spec, z_spec],
        out_specs=pl.BlockSpec((bsz, VOCAB_TILE), lambda j: (0, j)),
        out_shape=jax.ShapeDtypeStruct((bsz, vocab), jnp.float32),
    )(pairs, par, W, b2, logz)
    return out


# speedup vs baseline: 1.1041x; 1.1041x over previous
import functools

import jax
import jax.numpy as jnp
from jax import lax
from jax.experimental import pallas as pl
from jax.experimental.pallas import tpu as pltpu
from jax.experimental.pallas import tpu_sc as plsc

_NUM_SC = 2
_NUM_SUBCORES = 16
_NUM_WORKERS = _NUM_SC * _NUM_SUBCORES

VOCAB_TILE = 2048


def _sc_gather(table2, pair_idx):
    b, d2 = pair_idx.shape[0], table2.shape[1]
    b_per_w = b // _NUM_WORKERS
    mesh = plsc.VectorSubcoreMesh(core_axis_name="c", subcore_axis_name="s")

    @functools.partial(
        pl.kernel,
        mesh=mesh,
        out_type=jax.ShapeDtypeStruct((b, d2), jnp.float32),
        scratch_types=[
            pltpu.VMEM((b_per_w,), jnp.int32),
            pltpu.VMEM((b_per_w, d2), jnp.float32),
            pltpu.SemaphoreType.DMA,
        ],
    )
    def gather_kernel(table_hbm, idx_hbm, out_hbm, idx_v, rows_v, sem):
        wid = lax.axis_index("s") * _NUM_SC + lax.axis_index("c")
        base = wid * b_per_w
        pltpu.sync_copy(idx_hbm.at[pl.ds(base, b_per_w)], idx_v)
        pltpu.async_copy(table_hbm.at[idx_v], rows_v, sem).wait()
        pltpu.sync_copy(rows_v, out_hbm.at[pl.ds(base, b_per_w)])

    return gather_kernel(table2, pair_idx)


def _h_aug_bf16(e2_ref, par_ref, emb):
    e2 = e2_ref[...]
    h = jnp.where(par_ref[...] != 0, e2[:, emb:], e2[:, :emb])
    h = jnp.maximum(h, 0.0).astype(jnp.bfloat16)
    ones = jnp.ones((h.shape[0], 1), jnp.bfloat16)
    return jnp.concatenate([h, ones], axis=1)


def _stats_body(vocab, emb, e2_ref, par_ref, waugt_ref, logz_ref, m_s, s_s):
    j = pl.program_id(0)
    nt = pl.num_programs(0)

    @pl.when(j == 0)
    def _():
        m_s[...] = jnp.full(m_s.shape, -1e30, m_s.dtype)
        s_s[...] = jnp.zeros(s_s.shape, s_s.dtype)

    h_aug = _h_aug_bf16(e2_ref, par_ref, emb)
    logits = lax.dot_general(
        h_aug, waugt_ref[...].astype(jnp.bfloat16), (((1,), (0,)), ((), ())),
        preferred_element_type=jnp.float32)

    def _mask(x):
        col = j * VOCAB_TILE + lax.broadcasted_iota(
            jnp.int32, (1, VOCAB_TILE), 1)
        return jnp.where(col < vocab, x, -1e30)

    logits = lax.cond(j == nt - 1, _mask, lambda x: x, logits)

    m_tile = jnp.max(logits, axis=1, keepdims=True)
    m_old = m_s[...]
    m_new = jnp.maximum(m_old, m_tile)
    s_s[...] = s_s[...] * jnp.exp(m_old - m_new) + jnp.sum(
        jnp.exp(logits - m_new), axis=1, keepdims=True)
    m_s[...] = m_new

    @pl.when(j == nt - 1)
    def _():
        logz_ref[...] = m_new + jnp.log(s_s[...])


def _write_body(emb, e2_ref, par_ref, waug_ref, logzt_ref, outt_ref):
    h_aug = _h_aug_bf16(e2_ref, par_ref, emb)
    logits_t = lax.dot_general(
        waug_ref[...].astype(jnp.bfloat16), h_aug, (((1,), (1,)), ((), ())),
        preferred_element_type=jnp.float32)
    outt_ref[...] = logits_t - logzt_ref[...]


def kernel(input, table, W, b):
    bsz, emb = input.shape[0], table.shape[1]
    vocab = W.shape[0]

    idx = input.astype(jnp.int32)
    table2 = table.reshape(vocab // 2, 2 * emb)
    pairs = _sc_gather(table2, idx >> 1)
    par = (idx & 1).reshape(bsz, 1)

    w_aug = jnp.concatenate([W, b.reshape(vocab, 1)], axis=1)
    nt = pl.cdiv(vocab, VOCAB_TILE)
    e2_spec = pl.BlockSpec((bsz, 2 * emb), lambda j: (0, 0))
    par_spec = pl.BlockSpec((bsz, 1), lambda j: (0, 0))

    logz = pl.pallas_call(
        functools.partial(_stats_body, vocab, emb),
        grid=(nt,),
        in_specs=[
            e2_spec,
            par_spec,
            pl.BlockSpec((emb + 1, VOCAB_TILE), lambda j: (0, j)),
        ],
        out_specs=pl.BlockSpec((bsz, 1), lambda j: (0, 0)),
        out_shape=jax.ShapeDtypeStruct((bsz, 1), jnp.float32),
        scratch_shapes=[
            pltpu.VMEM((bsz, 1), jnp.float32),
            pltpu.VMEM((bsz, 1), jnp.float32),
        ],
    )(pairs, par, w_aug.T)

    outt = pl.pallas_call(
        functools.partial(_write_body, emb),
        grid=(nt,),
        in_specs=[
            e2_spec,
            par_spec,
            pl.BlockSpec((VOCAB_TILE, emb + 1), lambda j: (j, 0)),
            pl.BlockSpec((1, bsz), lambda j: (0, 0)),
        ],
        out_specs=pl.BlockSpec((VOCAB_TILE, bsz), lambda j: (j, 0)),
        out_shape=jax.ShapeDtypeStruct((vocab, bsz), jnp.float32),
    )(pairs, par, w_aug, logz.reshape(1, bsz))
    return outt.T


# speedup vs baseline: 1.4095x; 1.2766x over previous
import functools

import jax
import jax.numpy as jnp
from jax import lax
from jax.experimental import pallas as pl
from jax.experimental.pallas import tpu as pltpu
from jax.experimental.pallas import tpu_sc as plsc

_NUM_SC = 2
_NUM_SUBCORES = 16
_NUM_WORKERS = _NUM_SC * _NUM_SUBCORES

VOCAB_TILE = 2048


def _sc_gather(table, idx):
    b, d = idx.shape[0], table.shape[1]
    b_per_w = b // _NUM_WORKERS
    mesh = plsc.VectorSubcoreMesh(core_axis_name="c", subcore_axis_name="s")

    @functools.partial(
        pl.kernel,
        mesh=mesh,
        out_type=jax.ShapeDtypeStruct((b, d), jnp.float32),
        scratch_types=[
            pltpu.VMEM((b_per_w,), jnp.int32),
            pltpu.VMEM((b_per_w, d), jnp.float32),
            pltpu.SemaphoreType.DMA,
        ],
        compiler_params=pltpu.CompilerParams(use_tc_tiling_on_sc=False),
    )
    def gather_kernel(table_hbm, idx_hbm, out_hbm, idx_v, rows_v, sem):
        wid = lax.axis_index("s") * _NUM_SC + lax.axis_index("c")
        base = wid * b_per_w
        pltpu.sync_copy(idx_hbm.at[pl.ds(base, b_per_w)], idx_v)
        pltpu.async_copy(table_hbm.at[idx_v], rows_v, sem).wait()
        pltpu.sync_copy(rows_v, out_hbm.at[pl.ds(base, b_per_w)])

    return gather_kernel(table, idx)


def _h_aug_bf16(e_ref):
    h = jnp.maximum(e_ref[...], 0.0).astype(jnp.bfloat16)
    ones = jnp.ones((h.shape[0], 1), jnp.bfloat16)
    return jnp.concatenate([h, ones], axis=1)


def _stats_body(e_ref, waugt_ref, logz_ref, m_s, s_s):
    j = pl.program_id(0)
    nt = pl.num_programs(0)

    @pl.when(j == 0)
    def _():
        m_s[...] = jnp.full(m_s.shape, -1e30, m_s.dtype)
        s_s[...] = jnp.zeros(s_s.shape, s_s.dtype)

    h_aug = _h_aug_bf16(e_ref)
    logits = lax.dot_general(
        h_aug, waugt_ref[...].astype(jnp.bfloat16), (((1,), (0,)), ((), ())),
        preferred_element_type=jnp.float32).astype(jnp.bfloat16)

    m_tile = jnp.max(logits, axis=1, keepdims=True).astype(jnp.float32)
    m_old = m_s[...]
    m_new = jnp.maximum(m_old, m_tile)
    e = jnp.exp(logits - m_new.astype(jnp.bfloat16))
    ones_col = jnp.ones((VOCAB_TILE, 1), jnp.bfloat16)
    s_tile = lax.dot_general(
        e, ones_col, (((1,), (0,)), ((), ())),
        preferred_element_type=jnp.float32)
    s_s[...] = s_s[...] * jnp.exp(m_old - m_new) + s_tile
    m_s[...] = m_new

    @pl.when(j == nt - 1)
    def _():
        logz_ref[...] = m_new + jnp.log(s_s[...])


def _write_body(e_ref, waug_ref, logzt_ref, outt_ref):
    h_aug = _h_aug_bf16(e_ref)
    logits_t = lax.dot_general(
        waug_ref[...].astype(jnp.bfloat16), h_aug, (((1,), (1,)), ((), ())),
        preferred_element_type=jnp.float32)
    outt_ref[...] = logits_t - logzt_ref[...]


def kernel(input, table, W, b):
    bsz, emb = input.shape[0], table.shape[1]
    vocab = W.shape[0]

    idx = input.astype(jnp.int32)
    embeds = _sc_gather(table, idx)

    nt = pl.cdiv(vocab, VOCAB_TILE)
    npad = nt * VOCAB_TILE - vocab
    w_pad = jnp.concatenate([W, jnp.zeros((npad, emb), jnp.float32)], axis=0)
    b_pad = jnp.concatenate([b, jnp.full((npad,), -1e30, jnp.float32)])
    w_aug = jnp.concatenate([w_pad, b_pad.reshape(-1, 1)], axis=1)

    e_spec = pl.BlockSpec((bsz, emb), lambda j: (0, 0))

    logz = pl.pallas_call(
        _stats_body,
        grid=(nt,),
        in_specs=[
            e_spec,
            pl.BlockSpec((emb + 1, VOCAB_TILE), lambda j: (0, j)),
        ],
        out_specs=pl.BlockSpec((bsz, 1), lambda j: (0, 0)),
        out_shape=jax.ShapeDtypeStruct((bsz, 1), jnp.float32),
        scratch_shapes=[
            pltpu.VMEM((bsz, 1), jnp.float32),
            pltpu.VMEM((bsz, 1), jnp.float32),
        ],
    )(embeds, w_aug.T)

    outt = pl.pallas_call(
        _write_body,
        grid=(nt,),
        in_specs=[
            e_spec,
            pl.BlockSpec((VOCAB_TILE, emb + 1), lambda j: (j, 0)),
            pl.BlockSpec((1, bsz), lambda j: (0, 0)),
        ],
        out_specs=pl.BlockSpec((VOCAB_TILE, bsz), lambda j: (j, 0)),
        out_shape=jax.ShapeDtypeStruct((vocab, bsz), jnp.float32),
    )(embeds, w_aug, logz.reshape(1, bsz))
    return outt.T


# speedup vs baseline: 1.9800x; 1.4048x over previous
import functools

import jax
import jax.numpy as jnp
from jax import lax
from jax.experimental import pallas as pl
from jax.experimental.pallas import tpu as pltpu
from jax.experimental.pallas import tpu_sc as plsc

_NUM_SC = 2
_NUM_SUBCORES = 16
_NUM_WORKERS = _NUM_SC * _NUM_SUBCORES

VOCAB_TILE = 2048


def _sc_gather(table, idx):
    b, d = idx.shape[0], table.shape[1]
    b_per_w = b // _NUM_WORKERS
    mesh = plsc.VectorSubcoreMesh(core_axis_name="c", subcore_axis_name="s")

    @functools.partial(
        pl.kernel,
        mesh=mesh,
        out_type=jax.ShapeDtypeStruct((b, d), jnp.float32),
        scratch_types=[
            pltpu.VMEM((b_per_w,), jnp.int32),
            pltpu.VMEM((b_per_w, d), jnp.float32),
            pltpu.SemaphoreType.DMA,
        ],
        compiler_params=pltpu.CompilerParams(use_tc_tiling_on_sc=False),
    )
    def gather_kernel(table_hbm, idx_hbm, out_hbm, idx_v, rows_v, sem):
        wid = lax.axis_index("s") * _NUM_SC + lax.axis_index("c")
        base = wid * b_per_w
        pltpu.sync_copy(idx_hbm.at[pl.ds(base, b_per_w)], idx_v)
        pltpu.async_copy(table_hbm.at[idx_v], rows_v, sem).wait()
        pltpu.sync_copy(rows_v, out_hbm.at[pl.ds(base, b_per_w)])

    return gather_kernel(table, idx)


def _h_aug_bf16(e_ref):
    h = jnp.maximum(e_ref[...], 0.0).astype(jnp.bfloat16)
    ones = jnp.ones((h.shape[0], 1), jnp.bfloat16)
    return jnp.concatenate([h, ones], axis=1)


def _stats_body(e_ref, waugt_ref, logz_ref, m_s, s_s):
    j = pl.program_id(0)
    nt = pl.num_programs(0)

    @pl.when(j == 0)
    def _():
        m_s[...] = jnp.full(m_s.shape, -1e30, m_s.dtype)
        s_s[...] = jnp.zeros(s_s.shape, s_s.dtype)

    h_aug = _h_aug_bf16(e_ref)
    logits = lax.dot_general(
        h_aug, waugt_ref[...].astype(jnp.bfloat16), (((1,), (0,)), ((), ())),
        preferred_element_type=jnp.float32).astype(jnp.bfloat16)

    m_tile = jnp.max(logits, axis=1, keepdims=True).astype(jnp.float32)
    m_old = m_s[...]
    m_new = jnp.maximum(m_old, m_tile)
    e = jnp.exp(logits - m_new.astype(jnp.bfloat16))
    ones_col = jnp.ones((VOCAB_TILE, 1), jnp.bfloat16)
    s_tile = lax.dot_general(
        e, ones_col, (((1,), (0,)), ((), ())),
        preferred_element_type=jnp.float32)
    s_s[...] = s_s[...] * jnp.exp(m_old - m_new) + s_tile
    m_s[...] = m_new

    @pl.when(j == nt - 1)
    def _():
        logz_ref[...] = m_new + jnp.log(s_s[...])


def _write_body(e_ref, waugt_ref, logzt_ref, outt_ref):
    h_aug = _h_aug_bf16(e_ref)
    logits_t = lax.dot_general(
        waugt_ref[...].astype(jnp.bfloat16), h_aug, (((0,), (1,)), ((), ())),
        preferred_element_type=jnp.float32)
    outt_ref[...] = logits_t - logzt_ref[...]


def kernel(input, table, W, b):
    bsz, emb = input.shape[0], table.shape[1]
    vocab = W.shape[0]

    idx = input.astype(jnp.int32)
    embeds = _sc_gather(table, idx)

    nt = pl.cdiv(vocab, VOCAB_TILE)
    npad = nt * VOCAB_TILE - vocab
    wt_pad = jnp.pad(W.T, ((0, 0), (0, npad)))
    b_row = jnp.concatenate([b, jnp.full((npad,), -1e30, jnp.float32)])
    waug_t = jnp.concatenate([wt_pad, b_row.reshape(1, -1)], axis=0)

    e_spec = pl.BlockSpec((bsz, emb), lambda j: (0, 0))

    logz = pl.pallas_call(
        _stats_body,
        grid=(nt,),
        in_specs=[
            e_spec,
            pl.BlockSpec((emb + 1, VOCAB_TILE), lambda j: (0, j)),
        ],
        out_specs=pl.BlockSpec((bsz, 1), lambda j: (0, 0)),
        out_shape=jax.ShapeDtypeStruct((bsz, 1), jnp.float32),
        scratch_shapes=[
            pltpu.VMEM((bsz, 1), jnp.float32),
            pltpu.VMEM((bsz, 1), jnp.float32),
        ],
    )(embeds, waug_t)

    outt = pl.pallas_call(
        _write_body,
        grid=(nt,),
        in_specs=[
            e_spec,
            pl.BlockSpec((emb + 1, VOCAB_TILE), lambda j: (0, j)),
            pl.BlockSpec((1, bsz), lambda j: (0, 0)),
        ],
        out_specs=pl.BlockSpec((VOCAB_TILE, bsz), lambda j: (j, 0)),
        out_shape=jax.ShapeDtypeStruct((vocab, bsz), jnp.float32),
        compiler_params=pltpu.CompilerParams(fuse_transposed_lhs_in_matmul=True),
    )(embeds, waug_t, logz.reshape(1, bsz))
    return outt.T


# speedup vs baseline: 2.0947x; 1.0579x over previous
import functools

import jax
import jax.numpy as jnp
from jax import lax
from jax.experimental import pallas as pl
from jax.experimental.pallas import tpu as pltpu
from jax.experimental.pallas import tpu_sc as plsc

_NUM_SC = 2
_NUM_SUBCORES = 16
_NUM_WORKERS = _NUM_SC * _NUM_SUBCORES

VOCAB_TILE = 2048


def _sc_gather(table, idx):
    b, d = idx.shape[0], table.shape[1]
    b_per_w = b // _NUM_WORKERS
    mesh = plsc.VectorSubcoreMesh(core_axis_name="c", subcore_axis_name="s")

    @functools.partial(
        pl.kernel,
        mesh=mesh,
        out_type=jax.ShapeDtypeStruct((b, d), jnp.float32),
        scratch_types=[
            pltpu.VMEM((b_per_w,), jnp.int32),
            pltpu.VMEM((b_per_w, d), jnp.float32),
            pltpu.SemaphoreType.DMA,
        ],
        compiler_params=pltpu.CompilerParams(use_tc_tiling_on_sc=False),
    )
    def gather_kernel(table_hbm, idx_hbm, out_hbm, idx_v, rows_v, sem):
        wid = lax.axis_index("s") * _NUM_SC + lax.axis_index("c")
        base = wid * b_per_w
        pltpu.sync_copy(idx_hbm.at[pl.ds(base, b_per_w)], idx_v)
        pltpu.async_copy(table_hbm.at[idx_v], rows_v, sem).wait()
        pltpu.sync_copy(rows_v, out_hbm.at[pl.ds(base, b_per_w)])

    return gather_kernel(table, idx)


def _h_aug_bf16(e_ref):
    h = jnp.maximum(e_ref[...], 0.0).astype(jnp.bfloat16)
    ones = jnp.ones((h.shape[0], 1), jnp.bfloat16)
    return jnp.concatenate([h, ones], axis=1)


def _stats_body(e_ref, waugt_ref, logz_ref, m_s, s_s):
    j = pl.program_id(0)
    nt = pl.num_programs(0)

    @pl.when(j == 0)
    def _():
        m_s[...] = jnp.full(m_s.shape, -1e30, m_s.dtype)
        s_s[...] = jnp.zeros(s_s.shape, s_s.dtype)

    h_aug = _h_aug_bf16(e_ref)
    logits = lax.dot_general(
        h_aug, waugt_ref[...].astype(jnp.bfloat16), (((1,), (0,)), ((), ())),
        preferred_element_type=jnp.float32).astype(jnp.bfloat16)

    m_tile = jnp.max(logits, axis=1, keepdims=True).astype(jnp.float32)
    m_old = m_s[...]
    m_new = jnp.maximum(m_old, m_tile)
    e = jnp.exp(logits - m_new.astype(jnp.bfloat16))
    s_tile = jnp.sum(e, axis=1, keepdims=True).astype(jnp.float32)
    s_s[...] = s_s[...] * jnp.exp(m_old - m_new) + s_tile
    m_s[...] = m_new

    @pl.when(j == nt - 1)
    def _():
        logz_ref[...] = m_new + jnp.log(s_s[...])


def _write_body(e_ref, waugt_ref, logzt_ref, outt_ref):
    h_aug = _h_aug_bf16(e_ref)
    logits_t = lax.dot_general(
        waugt_ref[...].astype(jnp.bfloat16), h_aug, (((0,), (1,)), ((), ())),
        preferred_element_type=jnp.float32)
    outt_ref[...] = logits_t - logzt_ref[...]


def kernel(input, table, W, b):
    bsz, emb = input.shape[0], table.shape[1]
    vocab = W.shape[0]

    idx = input.astype(jnp.int32)
    embeds = _sc_gather(table, idx)

    nt = pl.cdiv(vocab, VOCAB_TILE)
    npad = nt * VOCAB_TILE - vocab
    wt_pad = jnp.pad(W.T, ((0, 0), (0, npad)))
    b_row = jnp.concatenate([b, jnp.full((npad,), -1e30, jnp.float32)])
    waug_t = jnp.concatenate([wt_pad, b_row.reshape(1, -1)], axis=0)

    e_spec = pl.BlockSpec((bsz, emb), lambda j: (0, 0))

    logz = pl.pallas_call(
        _stats_body,
        grid=(nt,),
        in_specs=[
            e_spec,
            pl.BlockSpec((emb + 1, VOCAB_TILE), lambda j: (0, j)),
        ],
        out_specs=pl.BlockSpec((bsz, 1), lambda j: (0, 0)),
        out_shape=jax.ShapeDtypeStruct((bsz, 1), jnp.float32),
        scratch_shapes=[
            pltpu.VMEM((bsz, 1), jnp.float32),
            pltpu.VMEM((bsz, 1), jnp.float32),
        ],
    )(embeds, waug_t)

    outt = pl.pallas_call(
        _write_body,
        grid=(nt,),
        in_specs=[
            e_spec,
            pl.BlockSpec((emb + 1, VOCAB_TILE), lambda j: (0, j)),
            pl.BlockSpec((1, bsz), lambda j: (0, 0)),
        ],
        out_specs=pl.BlockSpec((VOCAB_TILE, bsz), lambda j: (j, 0)),
        out_shape=jax.ShapeDtypeStruct((vocab, bsz), jnp.float32),
        compiler_params=pltpu.CompilerParams(fuse_transposed_lhs_in_matmul=True),
    )(embeds, waug_t, logz.reshape(1, bsz))
    return outt.T


# speedup vs baseline: 2.1128x; 1.0086x over previous
import functools

import jax
import jax.numpy as jnp
from jax import lax
from jax.experimental import pallas as pl
from jax.experimental.pallas import tpu as pltpu
from jax.experimental.pallas import tpu_sc as plsc

_NUM_SC = 2
_NUM_SUBCORES = 16
_NUM_WORKERS = _NUM_SC * _NUM_SUBCORES

VOCAB_TILE = 4096


def _sc_gather(table, idx):
    b, d = idx.shape[0], table.shape[1]
    b_per_w = b // _NUM_WORKERS
    mesh = plsc.VectorSubcoreMesh(core_axis_name="c", subcore_axis_name="s")

    @functools.partial(
        pl.kernel,
        mesh=mesh,
        out_type=jax.ShapeDtypeStruct((b, d), jnp.float32),
        scratch_types=[
            pltpu.VMEM((b_per_w,), jnp.int32),
            pltpu.VMEM((b_per_w, d), jnp.float32),
            pltpu.SemaphoreType.DMA,
        ],
        compiler_params=pltpu.CompilerParams(use_tc_tiling_on_sc=False),
    )
    def gather_kernel(table_hbm, idx_hbm, out_hbm, idx_v, rows_v, sem):
        wid = lax.axis_index("s") * _NUM_SC + lax.axis_index("c")
        base = wid * b_per_w
        pltpu.sync_copy(idx_hbm.at[pl.ds(base, b_per_w)], idx_v)
        pltpu.async_copy(table_hbm.at[idx_v], rows_v, sem).wait()
        pltpu.sync_copy(rows_v, out_hbm.at[pl.ds(base, b_per_w)])

    return gather_kernel(table, idx)


def _h_aug_bf16(e_ref):
    h = jnp.maximum(e_ref[...], 0.0).astype(jnp.bfloat16)
    ones = jnp.ones((h.shape[0], 1), jnp.bfloat16)
    return jnp.concatenate([h, ones], axis=1)


def _stats_body(e_ref, waugt_ref, logz_ref, m_s, s_s):
    j = pl.program_id(0)
    nt = pl.num_programs(0)

    @pl.when(j == 0)
    def _():
        m_s[...] = jnp.full(m_s.shape, -1e30, m_s.dtype)
        s_s[...] = jnp.zeros(s_s.shape, s_s.dtype)

    h_aug = _h_aug_bf16(e_ref)
    logits = lax.dot_general(
        h_aug, waugt_ref[...].astype(jnp.bfloat16), (((1,), (0,)), ((), ())),
        preferred_element_type=jnp.float32).astype(jnp.bfloat16)

    m_tile = jnp.max(logits, axis=1, keepdims=True).astype(jnp.float32)
    m_old = m_s[...]
    m_new = jnp.maximum(m_old, m_tile)
    e = jnp.exp(logits - m_new.astype(jnp.bfloat16))
    s_tile = jnp.sum(e, axis=1, keepdims=True).astype(jnp.float32)
    s_s[...] = s_s[...] * jnp.exp(m_old - m_new) + s_tile
    m_s[...] = m_new

    @pl.when(j == nt - 1)
    def _():
        logz_ref[...] = m_new + jnp.log(s_s[...])


def _write_body(e_ref, waugt_ref, logzt_ref, outt_ref):
    h_aug = _h_aug_bf16(e_ref)
    logits_t = lax.dot_general(
        waugt_ref[...].astype(jnp.bfloat16), h_aug, (((0,), (1,)), ((), ())),
        preferred_element_type=jnp.float32)
    outt_ref[...] = logits_t - logzt_ref[...]


def kernel(input, table, W, b):
    bsz, emb = input.shape[0], table.shape[1]
    vocab = W.shape[0]

    idx = input.astype(jnp.int32)
    embeds = _sc_gather(table, idx)

    nt = pl.cdiv(vocab, VOCAB_TILE)
    npad = nt * VOCAB_TILE - vocab
    wt_pad = jnp.pad(W.T, ((0, 0), (0, npad)))
    b_row = jnp.concatenate([b, jnp.full((npad,), -1e30, jnp.float32)])
    waug_t = jnp.concatenate([wt_pad, b_row.reshape(1, -1)], axis=0)

    e_spec = pl.BlockSpec((bsz, emb), lambda j: (0, 0))

    logz = pl.pallas_call(
        _stats_body,
        grid=(nt,),
        in_specs=[
            e_spec,
            pl.BlockSpec((emb + 1, VOCAB_TILE), lambda j: (0, j)),
        ],
        out_specs=pl.BlockSpec((bsz, 1), lambda j: (0, 0)),
        out_shape=jax.ShapeDtypeStruct((bsz, 1), jnp.float32),
        scratch_shapes=[
            pltpu.VMEM((bsz, 1), jnp.float32),
            pltpu.VMEM((bsz, 1), jnp.float32),
        ],
    )(embeds, waug_t)

    outt = pl.pallas_call(
        _write_body,
        grid=(nt,),
        in_specs=[
            e_spec,
            pl.BlockSpec((emb + 1, VOCAB_TILE), lambda j: (0, j)),
            pl.BlockSpec((1, bsz), lambda j: (0, 0)),
        ],
        out_specs=pl.BlockSpec((VOCAB_TILE, bsz), lambda j: (j, 0)),
        out_shape=jax.ShapeDtypeStruct((vocab, bsz), jnp.float32),
        compiler_params=pltpu.CompilerParams(fuse_transposed_lhs_in_matmul=True),
    )(embeds, waug_t, logz.reshape(1, bsz))
    return outt.T


# speedup vs baseline: 2.1842x; 1.0338x over previous
import functools

import jax
import jax.numpy as jnp
from jax import lax
from jax.experimental import pallas as pl
from jax.experimental.pallas import tpu as pltpu
from jax.experimental.pallas import tpu_sc as plsc

_NUM_SC = 2
_NUM_SUBCORES = 16
_NUM_WORKERS = _NUM_SC * _NUM_SUBCORES

VOCAB_TILE = 4096


def _sc_gather(table, idx):
    b, d = idx.shape[0], table.shape[1]
    b_per_w = b // _NUM_WORKERS
    mesh = plsc.VectorSubcoreMesh(core_axis_name="c", subcore_axis_name="s")

    @functools.partial(
        pl.kernel,
        mesh=mesh,
        out_type=jax.ShapeDtypeStruct((b, d), jnp.float32),
        scratch_types=[
            pltpu.VMEM((b_per_w,), jnp.int32),
            pltpu.VMEM((b_per_w, d), jnp.float32),
            pltpu.SemaphoreType.DMA,
        ],
        compiler_params=pltpu.CompilerParams(use_tc_tiling_on_sc=False),
    )
    def gather_kernel(table_hbm, idx_hbm, out_hbm, idx_v, rows_v, sem):
        wid = lax.axis_index("s") * _NUM_SC + lax.axis_index("c")
        base = wid * b_per_w
        pltpu.sync_copy(idx_hbm.at[pl.ds(base, b_per_w)], idx_v)
        pltpu.async_copy(table_hbm.at[idx_v], rows_v, sem).wait()
        pltpu.sync_copy(rows_v, out_hbm.at[pl.ds(base, b_per_w)])

    return gather_kernel(table, idx)


def _h_aug_bf16(e_ref):
    h = jnp.maximum(e_ref[...], 0.0).astype(jnp.bfloat16)
    ones = jnp.ones((h.shape[0], 1), jnp.bfloat16)
    return jnp.concatenate([h, ones], axis=1)


def _stats_body(e_ref, waugt_ref, logz_ref, m_s, s_s):
    j = pl.program_id(0)
    nt = pl.num_programs(0)

    @pl.when(j == 0)
    def _():
        m_s[...] = jnp.full(m_s.shape, -1e30, m_s.dtype)
        s_s[...] = jnp.zeros(s_s.shape, s_s.dtype)

    h_aug = _h_aug_bf16(e_ref)
    logits = lax.dot_general(
        h_aug, waugt_ref[...].astype(jnp.bfloat16), (((1,), (0,)), ((), ())),
        preferred_element_type=jnp.float32).astype(jnp.bfloat16)

    m_tile = jnp.max(logits, axis=1, keepdims=True).astype(jnp.float32)
    m_old = m_s[...]
    m_new = jnp.maximum(m_old, m_tile)
    e = jnp.exp(logits - m_new.astype(jnp.bfloat16))
    while e.shape[1] > 512:
        half = e.shape[1] // 2
        e = e[:, :half] + e[:, half:]
    s_tile = jnp.sum(e.astype(jnp.float32), axis=1, keepdims=True)
    s_s[...] = s_s[...] * jnp.exp(m_old - m_new) + s_tile
    m_s[...] = m_new

    @pl.when(j == nt - 1)
    def _():
        logz_ref[...] = m_new + jnp.log(s_s[...])


def _write_body(e_ref, waugt_ref, logzt_ref, outt_ref):
    h_aug = _h_aug_bf16(e_ref)
    logits_t = lax.dot_general(
        waugt_ref[...].astype(jnp.bfloat16), h_aug, (((0,), (1,)), ((), ())),
        preferred_element_type=jnp.float32)
    outt_ref[...] = logits_t - logzt_ref[...]


def kernel(input, table, W, b):
    bsz, emb = input.shape[0], table.shape[1]
    vocab = W.shape[0]

    idx = input.astype(jnp.int32)
    embeds = _sc_gather(table, idx)

    nt = pl.cdiv(vocab, VOCAB_TILE)
    npad = nt * VOCAB_TILE - vocab
    wt_pad = jnp.pad(W.T, ((0, 0), (0, npad)))
    b_row = jnp.concatenate([b, jnp.full((npad,), -1e30, jnp.float32)])
    waug_t = jnp.concatenate([wt_pad, b_row.reshape(1, -1)], axis=0)

    e_spec = pl.BlockSpec((bsz, emb), lambda j: (0, 0))

    logz = pl.pallas_call(
        _stats_body,
        grid=(nt,),
        in_specs=[
            e_spec,
            pl.BlockSpec((emb + 1, VOCAB_TILE), lambda j: (0, j)),
        ],
        out_specs=pl.BlockSpec((bsz, 1), lambda j: (0, 0)),
        out_shape=jax.ShapeDtypeStruct((bsz, 1), jnp.float32),
        scratch_shapes=[
            pltpu.VMEM((bsz, 1), jnp.float32),
            pltpu.VMEM((bsz, 1), jnp.float32),
        ],
    )(embeds, waug_t)

    outt = pl.pallas_call(
        _write_body,
        grid=(nt,),
        in_specs=[
            e_spec,
            pl.BlockSpec((emb + 1, VOCAB_TILE), lambda j: (0, j)),
            pl.BlockSpec((1, bsz), lambda j: (0, 0)),
        ],
        out_specs=pl.BlockSpec((VOCAB_TILE, bsz), lambda j: (j, 0)),
        out_shape=jax.ShapeDtypeStruct((vocab, bsz), jnp.float32),
        compiler_params=pltpu.CompilerParams(fuse_transposed_lhs_in_matmul=True),
    )(embeds, waug_t, logz.reshape(1, bsz))
    return outt.T


# speedup vs baseline: 2.2512x; 1.0307x over previous
import functools

import jax
import jax.numpy as jnp
from jax import lax
from jax.experimental import pallas as pl
from jax.experimental.pallas import tpu as pltpu
from jax.experimental.pallas import tpu_sc as plsc

_NUM_SC = 2
_NUM_SUBCORES = 16
_NUM_WORKERS = _NUM_SC * _NUM_SUBCORES

VOCAB_TILE = 4096


def _sc_gather(table, idx):
    b, d = idx.shape[0], table.shape[1]
    b_per_w = b // _NUM_WORKERS
    mesh = plsc.VectorSubcoreMesh(core_axis_name="c", subcore_axis_name="s")

    @functools.partial(
        pl.kernel,
        mesh=mesh,
        out_type=jax.ShapeDtypeStruct((b, d), jnp.float32),
        scratch_types=[
            pltpu.VMEM((b_per_w,), jnp.int32),
            pltpu.VMEM((b_per_w, d), jnp.float32),
            pltpu.SemaphoreType.DMA,
        ],
        compiler_params=pltpu.CompilerParams(use_tc_tiling_on_sc=False),
    )
    def gather_kernel(table_hbm, idx_hbm, out_hbm, idx_v, rows_v, sem):
        wid = lax.axis_index("s") * _NUM_SC + lax.axis_index("c")
        base = wid * b_per_w
        pltpu.sync_copy(idx_hbm.at[pl.ds(base, b_per_w)], idx_v)
        pltpu.async_copy(table_hbm.at[idx_v], rows_v, sem).wait()
        pltpu.sync_copy(rows_v, out_hbm.at[pl.ds(base, b_per_w)])

    return gather_kernel(table, idx)


def _h_aug_bf16(e_ref, emb):
    h = jnp.maximum(e_ref[...][:, :emb], 0.0).astype(jnp.bfloat16)
    ones = jnp.ones((h.shape[0], 1), jnp.bfloat16)
    return jnp.concatenate([h, ones], axis=1)


def _stats_body(emb, e_ref, waugt_ref, logz_ref, m_s, s_s):
    j = pl.program_id(0)
    nt = pl.num_programs(0)

    @pl.when(j == 0)
    def _():
        m_s[...] = jnp.full(m_s.shape, -1e30, m_s.dtype)
        s_s[...] = jnp.zeros(s_s.shape, s_s.dtype)

    h_aug = _h_aug_bf16(e_ref, emb)
    logits = lax.dot_general(
        h_aug, waugt_ref[...].astype(jnp.bfloat16), (((1,), (0,)), ((), ())),
        preferred_element_type=jnp.float32).astype(jnp.bfloat16)

    m_tile = jnp.max(logits, axis=1, keepdims=True).astype(jnp.float32)
    m_old = m_s[...]
    m_new = jnp.maximum(m_old, m_tile)
    e = jnp.exp(logits - m_new.astype(jnp.bfloat16))
    while e.shape[1] > 512:
        half = e.shape[1] // 2
        e = e[:, :half] + e[:, half:]
    s_tile = jnp.sum(e.astype(jnp.float32), axis=1, keepdims=True)
    s_s[...] = s_s[...] * jnp.exp(m_old - m_new) + s_tile
    m_s[...] = m_new

    @pl.when(j == nt - 1)
    def _():
        logz_ref[...] = m_new + jnp.log(s_s[...])


def _write_body(emb, e_ref, waugt_ref, logzt_ref, outt_ref):
    h_aug = _h_aug_bf16(e_ref, emb)
    logits_t = lax.dot_general(
        waugt_ref[...].astype(jnp.bfloat16), h_aug, (((0,), (1,)), ((), ())),
        preferred_element_type=jnp.float32)
    outt_ref[...] = logits_t - logzt_ref[...]


def kernel(input, table, W, b):
    bsz, emb = input.shape[0], table.shape[1]
    vocab = W.shape[0]

    idx = input.astype(jnp.int32)
    table_pad = jnp.pad(table, ((0, 0), (0, 128 - emb)))
    embeds = _sc_gather(table_pad, idx)

    nt = pl.cdiv(vocab, VOCAB_TILE)
    npad = nt * VOCAB_TILE - vocab
    wt_pad = jnp.pad(W.T, ((0, 0), (0, npad)))
    b_row = jnp.concatenate([b, jnp.full((npad,), -1e30, jnp.float32)])
    waug_t = jnp.concatenate([wt_pad, b_row.reshape(1, -1)], axis=0)

    e_spec = pl.BlockSpec((bsz, 128), lambda j: (0, 0))

    logz = pl.pallas_call(
        functools.partial(_stats_body, emb),
        grid=(nt,),
        in_specs=[
            e_spec,
            pl.BlockSpec((emb + 1, VOCAB_TILE), lambda j: (0, j)),
        ],
        out_specs=pl.BlockSpec((bsz, 1), lambda j: (0, 0)),
        out_shape=jax.ShapeDtypeStruct((bsz, 1), jnp.float32),
        scratch_shapes=[
            pltpu.VMEM((bsz, 1), jnp.float32),
            pltpu.VMEM((bsz, 1), jnp.float32),
        ],
    )(embeds, waug_t)

    outt = pl.pallas_call(
        functools.partial(_write_body, emb),
        grid=(nt,),
        in_specs=[
            e_spec,
            pl.BlockSpec((emb + 1, VOCAB_TILE), lambda j: (0, j)),
            pl.BlockSpec((1, bsz), lambda j: (0, 0)),
        ],
        out_specs=pl.BlockSpec((VOCAB_TILE, bsz), lambda j: (j, 0)),
        out_shape=jax.ShapeDtypeStruct((vocab, bsz), jnp.float32),
        compiler_params=pltpu.CompilerParams(fuse_transposed_lhs_in_matmul=True),
    )(embeds, waug_t, logz.reshape(1, bsz))
    return outt.T
